# baseline (speedup 1.0000x reference)
import jax
import jax.numpy as jnp
from jax import lax
from jax.experimental import pallas as pl
from jax.experimental.pallas import tpu as pltpu
from jax.experimental.pallas import tpu_sc as plsc

N_NODE = 50000
E = 800000
D_IN = 128
H = 64
HALF = H // 2
NC = 2
NS = 16
NW = NC * NS
SIB = 40
SETS = 10
EROWS = E // SIB
RPS = EROWS // NS
NSTEPS = RPS // SETS
ROWS_PT = N_NODE // NS
FCH = 40

EPW = E // NW
CCH = 1000

_sc_params = pltpu.CompilerParams(use_tc_tiling_on_sc=False,
                                  needs_layout_passes=False)
_sc_mesh = dict(core_axis_name="c", subcore_axis_name="s")


def _seg_round(h_lo, h_hi, src2d, dst2d, zrows, out_hbm, acc,
               sd, rows, isem, gsem, ssem, c, s):
    pltpu.sync_copy(zrows, rows.at[0])
    for j in range(ROWS_PT // FCH):
        pltpu.sync_copy(rows.at[0],
                        acc.at[pl.ds(s * ROWS_PT + j * FCH, FCH)])
    pltpu.sync_copy(rows.at[0, pl.ds(0, 5)],
                    acc.at[pl.ds(s * ROWS_PT + 3120, 5)])
    plsc.subcore_barrier()

    def _accum(h_half):
        base = s * RPS

        def load_idx(row, p):
            pltpu.async_copy(src2d.at[pl.ds(row, 1)],
                             sd.at[p, pl.ds(0, 1)], isem.at[p])
            pltpu.async_copy(dst2d.at[pl.ds(row, 1)],
                             sd.at[p, pl.ds(1, 1)], isem.at[p])

        def wait_idx(row, p):
            pltpu.make_async_copy(src2d.at[pl.ds(row, 1)],
                                  sd.at[p, pl.ds(0, 1)], isem.at[p]).wait()
            pltpu.make_async_copy(dst2d.at[pl.ds(row, 1)],
                                  sd.at[p, pl.ds(1, 1)], isem.at[p]).wait()

        def fire_gather(p):
            pltpu.async_copy(h_half.at[sd.at[p, 0]], rows.at[p], gsem.at[p])

        def wait_gather(p):
            pltpu.make_async_copy(h_half.at[sd.at[p, 0]], rows.at[p],
                                  gsem.at[p]).wait()

        def fire_scatter(p):
            pltpu.async_copy(rows.at[p], acc.at[sd.at[p, 1]], ssem.at[p],
                             add=True)

        def drain_scatter(p):
            pltpu.make_async_copy(rows.at[p], acc.at[sd.at[p, 1]],
                                  ssem.at[p]).wait()

        def step(j, carry):
            row0 = base + j * SETS
            for p in range(SETS):
                @pl.when(j > 0)
                def _(p=p):
                    drain_scatter(p)
                load_idx(row0 + p, p)
            for p in range(SETS):
                wait_idx(row0 + p, p)
                fire_gather(p)
            for p in range(SETS):
                wait_gather(p)
                fire_scatter(p)
            return carry
        lax.fori_loop(0, NSTEPS, step, 0)
        for p in range(SETS):
            drain_scatter(p)

    @pl.when(c == 0)
    def _lo():
        _accum(h_lo)

    @pl.when(c == 1)
    def _hi():
        _accum(h_hi)

    plsc.subcore_barrier()

    for j in range(ROWS_PT // FCH):
        r0 = s * ROWS_PT + j * FCH
        pltpu.sync_copy(acc.at[pl.ds(r0, FCH)], rows.at[0])
        pltpu.sync_copy(rows.at[0],
                        out_hbm.at[pl.ds(r0, FCH), pl.ds(c * HALF, HALF)])
    r0 = s * ROWS_PT + 3120
    pltpu.sync_copy(acc.at[pl.ds(r0, 5)], rows.at[0, pl.ds(0, 5)])
    pltpu.sync_copy(rows.at[0, pl.ds(0, 5)],
                    out_hbm.at[pl.ds(r0, 5), pl.ds(c * HALF, HALF)])


def _seg_scratch():
    return [
        pltpu.VMEM_SHARED((N_NODE, HALF), jnp.float32),
        pltpu.VMEM((SETS, 2, SIB), jnp.int32),
        pltpu.VMEM((SETS, SIB, HALF), jnp.float32),
        pltpu.SemaphoreType.DMA((SETS,)),
        pltpu.SemaphoreType.DMA((SETS,)),
        pltpu.SemaphoreType.DMA((SETS,)),
    ]


def _segsum1_body(h_lo, h_hi, src2d, dst2d, zrows, sum_out, *scr):
    c = lax.axis_index("c")
    s = lax.axis_index("s")
    _seg_round(h_lo, h_hi, src2d, dst2d, zrows, sum_out, *scr, c, s)


_segsum1 = pl.kernel(
    _segsum1_body,
    out_type=jax.ShapeDtypeStruct((N_NODE, H), jnp.float32),
    mesh=plsc.VectorSubcoreMesh(**_sc_mesh),
    scratch_types=_seg_scratch(),
    compiler_params=_sc_params,
)


def _cnt_body(edges_r, edges_v, zcnt, cntr_out, cntv_out, cnt_v, idx_v):
    c = lax.axis_index("c")
    s = lax.axis_index("s")
    w = s * NC + c
    ones16 = jnp.full((16,), 1.0, jnp.float32)
    ngrp = CCH // 16
    rem = CCH - ngrp * 16
    tailmask = lax.iota(jnp.int32, 16) >= (16 - rem)

    def _hist(edges, out):
        pltpu.sync_copy(zcnt, cnt_v)

        def body(i, carry):
            pltpu.sync_copy(edges.at[1, pl.ds(w * EPW + i * CCH, CCH)],
                            idx_v)
            for t in range(ngrp):
                idx16 = idx_v[pl.ds(t * 16, 16)]
                plsc.addupdate_scatter(cnt_v, [idx16], ones16)
            idxt = idx_v[pl.ds(CCH - 16, 16)]
            plsc.addupdate_scatter(cnt_v, [idxt], ones16, mask=tailmask)
            return carry
        lax.fori_loop(0, EPW // CCH, body, 0)

        pltpu.sync_copy(cnt_v, out.at[w])

    _hist(edges_r, cntr_out)
    _hist(edges_v, cntv_out)


_cnt_kernel = pl.kernel(
    _cnt_body,
    out_type=(jax.ShapeDtypeStruct((NW, N_NODE), jnp.float32),
              jax.ShapeDtypeStruct((NW, N_NODE), jnp.float32)),
    mesh=plsc.VectorSubcoreMesh(**_sc_mesh),
    scratch_types=[
        pltpu.VMEM((N_NODE,), jnp.float32),
        pltpu.VMEM((CCH,), jnp.int32),
    ],
    compiler_params=_sc_params,
)


_RB = 2000


def _proj_relu(x, W, b):
    n, d = x.shape
    h = W.shape[0]

    def body(x_ref, w_ref, b_ref, o_ref):
        y = jnp.dot(x_ref[...], w_ref[...].T,
                    preferred_element_type=jnp.float32) + b_ref[...]
        o_ref[...] = jnp.maximum(y, 0.0)

    return pl.pallas_call(
        body,
        grid=(n // _RB,),
        in_specs=[
            pl.BlockSpec((_RB, d), lambda i: (i, 0)),
            pl.BlockSpec((h, d), lambda i: (0, 0)),
            pl.BlockSpec((1, h), lambda i: (0, 0)),
        ],
        out_specs=pl.BlockSpec((_RB, h), lambda i: (i, 0)),
        out_shape=jax.ShapeDtypeStruct((n, h), jnp.float32),
    )(x, W, b.reshape(1, h))


def _combine(sums, cntT, xdst, Wl, bl, Wr, relu, Wout=None, bout=None):
    n = xdst.shape[0]
    nb = n // _RB
    out_h = 2 if Wout is not None else H

    def body(s_ref, cp_ref, xd_ref, wl_ref, bl_ref, wr_ref, *rest):
        if Wout is not None:
            wo_ref, bo_ref, o_ref = rest
        else:
            (o_ref,) = rest
        cnt = jnp.sum(cp_ref[...], axis=1)
        inv = 1.0 / jnp.maximum(cnt, 1.0)
        sm = s_ref[...] * inv[:, None]
        y = (jnp.dot(sm, wl_ref[...].T, preferred_element_type=jnp.float32)
             + bl_ref[...]
             + jnp.dot(xd_ref[...], wr_ref[...].T,
                       preferred_element_type=jnp.float32))
        if relu:
            y = jnp.maximum(y, 0.0)
        if Wout is not None:
            y = jnp.dot(y, wo_ref[...].T,
                        preferred_element_type=jnp.float32) + bo_ref[...]
        o_ref[...] = y

    in_specs = [
        pl.BlockSpec((_RB, H), lambda i: (i, 0)),
        pl.BlockSpec((_RB, NW), lambda i: (i, 0)),
        pl.BlockSpec((_RB, H), lambda i: (i, 0)),
        pl.BlockSpec((H, H), lambda i: (0, 0)),
        pl.BlockSpec((1, H), lambda i: (0, 0)),
        pl.BlockSpec((H, H), lambda i: (0, 0)),
    ]
    args = [sums, cntT, xdst, Wl, bl.reshape(1, H), Wr]
    if Wout is not None:
        in_specs += [
            pl.BlockSpec((2, H), lambda i: (0, 0)),
            pl.BlockSpec((1, 2), lambda i: (0, 0)),
        ]
        args += [Wout, bout.reshape(1, 2)]

    return pl.pallas_call(
        body,
        grid=(nb,),
        in_specs=in_specs,
        out_specs=pl.BlockSpec((_RB, out_h), lambda i: (i, 0)),
        out_shape=jax.ShapeDtypeStruct((n, out_h), jnp.float32),
    )(*args)


def kernel(x_user, x_article, edge_index_reads, edge_index_rev, W_in_user,
           b_in_user, W_in_article, b_in_article, Wl1_reads, bl1_reads,
           Wr1_reads, Wl1_rev, bl1_rev, Wr1_rev, Wl2_reads, bl2_reads,
           Wr2_reads, Wl2_rev, bl2_rev, Wr2_rev, W_out, b_out):
    srcr2d = edge_index_reads[0].reshape(EROWS, SIB)
    dstr2d = edge_index_reads[1].reshape(EROWS, SIB)
    srcv2d = edge_index_rev[0].reshape(EROWS, SIB)
    dstv2d = edge_index_rev[1].reshape(EROWS, SIB)

    zcnt = jnp.zeros((N_NODE,), jnp.float32)
    zrows = jnp.zeros((FCH, HALF), jnp.float32)
    cntr, cntv = _cnt_kernel(edge_index_reads, edge_index_rev, zcnt)
    cntrT, cntvT = cntr.T, cntv.T

    h_u = _proj_relu(x_user, W_in_user, b_in_user)
    h_a = _proj_relu(x_article, W_in_article, b_in_article)

    sum1a = _segsum1(h_u[:, :HALF], h_u[:, HALF:], srcr2d, dstr2d, zrows)
    sum1u = _segsum1(h_a[:, :HALF], h_a[:, HALF:], srcv2d, dstv2d, zrows)
    h1_a = _combine(sum1a, cntrT, h_a, Wl1_reads, bl1_reads, Wr1_reads, True)
    h1_u = _combine(sum1u, cntvT, h_u, Wl1_rev, bl1_rev, Wr1_rev, True)

    sum2a = _segsum1(h1_u[:, :HALF], h1_u[:, HALF:], srcr2d, dstr2d, zrows)
    out = _combine(sum2a, cntrT, h1_a, Wl2_reads, bl2_reads, Wr2_reads,
                   False, Wout=W_out, bout=b_out)
    return out


# speedup vs baseline: 1.1016x; 1.1016x over previous
import jax
import jax.numpy as jnp
from jax import lax
from jax.experimental import pallas as pl
from jax.experimental.pallas import tpu as pltpu
from jax.experimental.pallas import tpu_sc as plsc

N_NODE = 50000
E = 800000
D_IN = 128
H = 64
HALF = H // 2
NC = 2
NS = 16
NW = NC * NS
SIB = 80
SETS = 5
EROWS = E // SIB
RPS = EROWS // NS
NSTEPS = RPS // SETS
ROWS_PT = N_NODE // NS
FCH = 80

EPW = E // NW
CCH = 1000

_sc_params = pltpu.CompilerParams(use_tc_tiling_on_sc=False,
                                  needs_layout_passes=False)
_sc_mesh = dict(core_axis_name="c", subcore_axis_name="s")


def _seg_round(h_lo, h_hi, src2d, dst2d, zrows, out_hbm, acc,
               sd, rows, isem, gsem, ssem, c, s):
    pltpu.sync_copy(zrows, rows.at[0])
    for j in range(ROWS_PT // FCH):
        pltpu.sync_copy(rows.at[0],
                        acc.at[pl.ds(s * ROWS_PT + j * FCH, FCH)])
    pltpu.sync_copy(rows.at[0, pl.ds(0, 5)],
                    acc.at[pl.ds(s * ROWS_PT + 3120, 5)])
    plsc.subcore_barrier()

    def _accum(h_half):
        base = s * RPS

        def load_idx(row, p):
            pltpu.async_copy(src2d.at[pl.ds(row, 1)],
                             sd.at[p, pl.ds(0, 1)], isem.at[p])
            pltpu.async_copy(dst2d.at[pl.ds(row, 1)],
                             sd.at[p, pl.ds(1, 1)], isem.at[p])

        def wait_idx(row, p):
            pltpu.make_async_copy(src2d.at[pl.ds(row, 1)],
                                  sd.at[p, pl.ds(0, 1)], isem.at[p]).wait()
            pltpu.make_async_copy(dst2d.at[pl.ds(row, 1)],
                                  sd.at[p, pl.ds(1, 1)], isem.at[p]).wait()

        def fire_gather(p):
            pltpu.async_copy(h_half.at[sd.at[p, 0]], rows.at[p], gsem.at[p])

        def wait_gather(p):
            pltpu.make_async_copy(h_half.at[sd.at[p, 0]], rows.at[p],
                                  gsem.at[p]).wait()

        def fire_scatter(p):
            pltpu.async_copy(rows.at[p], acc.at[sd.at[p, 1]], ssem.at[p],
                             add=True)

        def drain_scatter(p):
            pltpu.make_async_copy(rows.at[p], acc.at[sd.at[p, 1]],
                                  ssem.at[p]).wait()

        def step(j, carry):
            row0 = base + j * SETS
            for p in range(SETS):
                @pl.when(j > 0)
                def _(p=p):
                    drain_scatter(p)
                load_idx(row0 + p, p)
            for p in range(SETS):
                wait_idx(row0 + p, p)
                fire_gather(p)
            for p in range(SETS):
                wait_gather(p)
                fire_scatter(p)
            return carry
        lax.fori_loop(0, NSTEPS, step, 0)
        for p in range(SETS):
            drain_scatter(p)

    @pl.when(c == 0)
    def _lo():
        _accum(h_lo)

    @pl.when(c == 1)
    def _hi():
        _accum(h_hi)

    plsc.subcore_barrier()

    for j in range(ROWS_PT // FCH):
        r0 = s * ROWS_PT + j * FCH
        pltpu.sync_copy(acc.at[pl.ds(r0, FCH)], rows.at[0])
        pltpu.sync_copy(rows.at[0],
                        out_hbm.at[pl.ds(r0, FCH), pl.ds(c * HALF, HALF)])
    r0 = s * ROWS_PT + 3120
    pltpu.sync_copy(acc.at[pl.ds(r0, 5)], rows.at[0, pl.ds(0, 5)])
    pltpu.sync_copy(rows.at[0, pl.ds(0, 5)],
                    out_hbm.at[pl.ds(r0, 5), pl.ds(c * HALF, HALF)])


def _seg_scratch():
    return [
        pltpu.VMEM_SHARED((N_NODE, HALF), jnp.float32),
        pltpu.VMEM((SETS, 2, SIB), jnp.int32),
        pltpu.VMEM((SETS, SIB, HALF), jnp.float32),
        pltpu.SemaphoreType.DMA((SETS,)),
        pltpu.SemaphoreType.DMA((SETS,)),
        pltpu.SemaphoreType.DMA((SETS,)),
    ]


def _segsum1_body(h_lo, h_hi, src2d, dst2d, zrows, sum_out, *scr):
    c = lax.axis_index("c")
    s = lax.axis_index("s")
    _seg_round(h_lo, h_hi, src2d, dst2d, zrows, sum_out, *scr, c, s)


_segsum1 = pl.kernel(
    _segsum1_body,
    out_type=jax.ShapeDtypeStruct((N_NODE, H), jnp.float32),
    mesh=plsc.VectorSubcoreMesh(**_sc_mesh),
    scratch_types=_seg_scratch(),
    compiler_params=_sc_params,
)


def _cnt_body(edges_r, edges_v, zcnt, cntr_out, cntv_out, cnt_v, idx_v):
    c = lax.axis_index("c")
    s = lax.axis_index("s")
    w = s * NC + c
    ones16 = jnp.full((16,), 1.0, jnp.float32)
    ngrp = CCH // 16
    rem = CCH - ngrp * 16
    tailmask = lax.iota(jnp.int32, 16) >= (16 - rem)

    def _hist(edges, out):
        pltpu.sync_copy(zcnt, cnt_v)

        def body(i, carry):
            pltpu.sync_copy(edges.at[1, pl.ds(w * EPW + i * CCH, CCH)],
                            idx_v)
            for t in range(ngrp):
                idx16 = idx_v[pl.ds(t * 16, 16)]
                plsc.addupdate_scatter(cnt_v, [idx16], ones16)
            idxt = idx_v[pl.ds(CCH - 16, 16)]
            plsc.addupdate_scatter(cnt_v, [idxt], ones16, mask=tailmask)
            return carry
        lax.fori_loop(0, EPW // CCH, body, 0)

        pltpu.sync_copy(cnt_v, out.at[w])

    _hist(edges_r, cntr_out)
    _hist(edges_v, cntv_out)


_cnt_kernel = pl.kernel(
    _cnt_body,
    out_type=(jax.ShapeDtypeStruct((NW, N_NODE), jnp.float32),
              jax.ShapeDtypeStruct((NW, N_NODE), jnp.float32)),
    mesh=plsc.VectorSubcoreMesh(**_sc_mesh),
    scratch_types=[
        pltpu.VMEM((N_NODE,), jnp.float32),
        pltpu.VMEM((CCH,), jnp.int32),
    ],
    compiler_params=_sc_params,
)


_RB = 2000


def _proj_relu(x, W, b):
    n, d = x.shape
    h = W.shape[0]

    def body(x_ref, w_ref, b_ref, o_ref):
        y = jnp.dot(x_ref[...], w_ref[...].T,
                    preferred_element_type=jnp.float32) + b_ref[...]
        o_ref[...] = jnp.maximum(y, 0.0)

    return pl.pallas_call(
        body,
        grid=(n // _RB,),
        in_specs=[
            pl.BlockSpec((_RB, d), lambda i: (i, 0)),
            pl.BlockSpec((h, d), lambda i: (0, 0)),
            pl.BlockSpec((1, h), lambda i: (0, 0)),
        ],
        out_specs=pl.BlockSpec((_RB, h), lambda i: (i, 0)),
        out_shape=jax.ShapeDtypeStruct((n, h), jnp.float32),
    )(x, W, b.reshape(1, h))


def _combine(sums, cntT, xdst, Wl, bl, Wr, relu, Wout=None, bout=None):
    n = xdst.shape[0]
    nb = n // _RB
    out_h = 2 if Wout is not None else H

    def body(s_ref, cp_ref, xd_ref, wl_ref, bl_ref, wr_ref, *rest):
        if Wout is not None:
            wo_ref, bo_ref, o_ref = rest
        else:
            (o_ref,) = rest
        cnt = jnp.sum(cp_ref[...], axis=1)
        inv = 1.0 / jnp.maximum(cnt, 1.0)
        sm = s_ref[...] * inv[:, None]
        y = (jnp.dot(sm, wl_ref[...].T, preferred_element_type=jnp.float32)
             + bl_ref[...]
             + jnp.dot(xd_ref[...], wr_ref[...].T,
                       preferred_element_type=jnp.float32))
        if relu:
            y = jnp.maximum(y, 0.0)
        if Wout is not None:
            y = jnp.dot(y, wo_ref[...].T,
                        preferred_element_type=jnp.float32) + bo_ref[...]
        o_ref[...] = y

    in_specs = [
        pl.BlockSpec((_RB, H), lambda i: (i, 0)),
        pl.BlockSpec((_RB, NW), lambda i: (i, 0)),
        pl.BlockSpec((_RB, H), lambda i: (i, 0)),
        pl.BlockSpec((H, H), lambda i: (0, 0)),
        pl.BlockSpec((1, H), lambda i: (0, 0)),
        pl.BlockSpec((H, H), lambda i: (0, 0)),
    ]
    args = [sums, cntT, xdst, Wl, bl.reshape(1, H), Wr]
    if Wout is not None:
        in_specs += [
            pl.BlockSpec((2, H), lambda i: (0, 0)),
            pl.BlockSpec((1, 2), lambda i: (0, 0)),
        ]
        args += [Wout, bout.reshape(1, 2)]

    return pl.pallas_call(
        body,
        grid=(nb,),
        in_specs=in_specs,
        out_specs=pl.BlockSpec((_RB, out_h), lambda i: (i, 0)),
        out_shape=jax.ShapeDtypeStruct((n, out_h), jnp.float32),
    )(*args)


def kernel(x_user, x_article, edge_index_reads, edge_index_rev, W_in_user,
           b_in_user, W_in_article, b_in_article, Wl1_reads, bl1_reads,
           Wr1_reads, Wl1_rev, bl1_rev, Wr1_rev, Wl2_reads, bl2_reads,
           Wr2_reads, Wl2_rev, bl2_rev, Wr2_rev, W_out, b_out):
    srcr2d = edge_index_reads[0].reshape(EROWS, SIB)
    dstr2d = edge_index_reads[1].reshape(EROWS, SIB)
    srcv2d = edge_index_rev[0].reshape(EROWS, SIB)
    dstv2d = edge_index_rev[1].reshape(EROWS, SIB)

    zcnt = jnp.zeros((N_NODE,), jnp.float32)
    zrows = jnp.zeros((FCH, HALF), jnp.float32)
    cntr, cntv = _cnt_kernel(edge_index_reads, edge_index_rev, zcnt)
    cntrT, cntvT = cntr.T, cntv.T

    h_u = _proj_relu(x_user, W_in_user, b_in_user)
    h_a = _proj_relu(x_article, W_in_article, b_in_article)

    sum1a = _segsum1(h_u[:, :HALF], h_u[:, HALF:], srcr2d, dstr2d, zrows)
    sum1u = _segsum1(h_a[:, :HALF], h_a[:, HALF:], srcv2d, dstv2d, zrows)
    h1_a = _combine(sum1a, cntrT, h_a, Wl1_reads, bl1_reads, Wr1_reads, True)
    h1_u = _combine(sum1u, cntvT, h_u, Wl1_rev, bl1_rev, Wr1_rev, True)

    sum2a = _segsum1(h1_u[:, :HALF], h1_u[:, HALF:], srcr2d, dstr2d, zrows)
    out = _combine(sum2a, cntrT, h1_a, Wl2_reads, bl2_reads, Wr2_reads,
                   False, Wout=W_out, bout=b_out)
    return out
